# Initial kernel scaffold; baseline (speedup 1.0000x reference)
import jax
import jax.numpy as jnp
from jax.experimental import pallas as pl


def kernel(outputs, targets):
    raise NotImplementedError("write your pallas kernel here")


# speedup vs baseline: 9.0644x; 9.0644x over previous
import functools

import jax
import jax.numpy as jnp
from jax.experimental import pallas as pl
from jax.experimental.pallas import tpu as pltpu

_SMOOTHING = 0.1
_CONFIDENCE = 1.0 - _SMOOTHING


def _row_pass_body(x_ref, t_ref, o_ref, m_ref, s_ref, sx_ref, xt_ref, *,
                   cc, num_classes):
    j = pl.program_id(1)

    @pl.when(j == 0)
    def _init():
        m_ref[...] = jnp.full_like(m_ref, -1e30)
        s_ref[...] = jnp.zeros_like(s_ref)
        sx_ref[...] = jnp.zeros_like(sx_ref)
        xt_ref[...] = jnp.zeros_like(xt_ref)

    x = x_ref[...]
    r = x.shape[0]
    bm = jnp.max(x, axis=1, keepdims=True)
    m_old = m_ref[...]
    m_new = jnp.maximum(m_old, bm)
    s_ref[...] = (s_ref[...] * jnp.exp(m_old - m_new)
                  + jnp.sum(jnp.exp(x - m_new), axis=1, keepdims=True))
    m_ref[...] = m_new
    sx_ref[...] = sx_ref[...] + jnp.sum(x, axis=1, keepdims=True)

    cols = j * cc + jax.lax.broadcasted_iota(jnp.int32, (r, cc), 1)
    xt_ref[...] = xt_ref[...] + jnp.sum(
        jnp.where(cols == t_ref[...], x, 0.0), axis=1, keepdims=True)

    @pl.when(j == pl.num_programs(1) - 1)
    def _finish():
        eps = _SMOOTHING / (num_classes - 1)
        lse = m_ref[...] + jnp.log(s_ref[...])
        o_ref[...] = (eps * (num_classes * lse - sx_ref[...])
                      + (_CONFIDENCE - eps) * (lse - xt_ref[...]))


def _mean_body(r_ref, o_ref):
    n = r_ref.shape[0]
    o_ref[...] = jnp.sum(r_ref[...], keepdims=True) * (1.0 / n)


def _pick_col_block(c, cap=6400):
    if c <= cap:
        return c
    best = 128
    for k in range(128, cap + 1, 128):
        if c % k == 0:
            best = k
    return best


def kernel(outputs, targets):
    n, c = outputs.shape
    r = 256 if n % 256 == 0 else n
    cc = _pick_col_block(c)
    t2 = targets.reshape(n, 1)

    row_losses = pl.pallas_call(
        functools.partial(_row_pass_body, cc=cc, num_classes=c),
        grid=(n // r, c // cc),
        in_specs=[
            pl.BlockSpec((r, cc), lambda i, j: (i, j)),
            pl.BlockSpec((r, 1), lambda i, j: (i, 0)),
        ],
        out_specs=pl.BlockSpec((r, 1), lambda i, j: (i, 0)),
        out_shape=jax.ShapeDtypeStruct((n, 1), jnp.float32),
        scratch_shapes=[
            pltpu.VMEM((r, 1), jnp.float32),
            pltpu.VMEM((r, 1), jnp.float32),
            pltpu.VMEM((r, 1), jnp.float32),
            pltpu.VMEM((r, 1), jnp.float32),
        ],
        compiler_params=pltpu.CompilerParams(
            dimension_semantics=("parallel", "arbitrary"),
        ),
    )(outputs, t2)

    loss = pl.pallas_call(
        _mean_body,
        out_shape=jax.ShapeDtypeStruct((1, 1), jnp.float32),
    )(row_losses)
    return loss[0, 0]
